# Initial kernel scaffold; baseline (speedup 1.0000x reference)
import jax
import jax.numpy as jnp
from jax.experimental import pallas as pl


def kernel(prediction, frame_h, frame_w):
    raise NotImplementedError("write your pallas kernel here")


# speedup vs baseline: 6.8505x; 6.8505x over previous
import functools

import jax
import jax.numpy as jnp
from jax import lax
from jax.experimental import pallas as pl
from jax.experimental.pallas import tpu as pltpu

_MIN_CONF = 0.25
_IOU_THR = 0.45
_N = 20000
_K = 1000
_KP = 1024
_GAIN = min(640 / 1080, 640 / 1920)
_PAD_X = (640 - 1920 * _GAIN) / 2.0
_PAD_Y = (640 - 1080 * _GAIN) / 2.0


def _phase_a_body(predT_ref, scores_ref, cid_ref, ltrb_ref):
    cls = predT_ref[5:85, :]
    m = jnp.max(cls, axis=0, keepdims=True)
    i0 = lax.broadcasted_iota(jnp.int32, (80, _N), 0)
    cid = jnp.min(jnp.where(cls == m, i0, jnp.int32(2**30)), axis=0,
                  keepdims=True)
    obj = predT_ref[4:5, :]
    conf = obj * m
    cvalid = (cid <= 3) | (cid == 5) | (cid == 7)
    keep = (conf >= _MIN_CONF) & cvalid
    scores_ref[...] = jnp.where(keep, conf, -1.0)
    cid_ref[...] = cid
    x = predT_ref[0:1, :]
    y = predT_ref[1:2, :]
    w = predT_ref[2:3, :]
    h = predT_ref[3:4, :]
    ltrb_ref[0:1, :] = jnp.round((x - w / 2.0 - _PAD_X) / _GAIN)
    ltrb_ref[1:2, :] = jnp.round((y - h / 2.0 - _PAD_Y) / _GAIN)
    ltrb_ref[2:3, :] = jnp.round((x + w / 2.0 - _PAD_X) / _GAIN)
    ltrb_ref[3:4, :] = jnp.round((y + h / 2.0 - _PAD_Y) / _GAIN)


def _phase_a(predT):
    return pl.pallas_call(
        _phase_a_body,
        out_shape=[
            jax.ShapeDtypeStruct((1, _N), jnp.float32),
            jax.ShapeDtypeStruct((1, _N), jnp.int32),
            jax.ShapeDtypeStruct((4, _N), jnp.float32),
        ],
    )(predT)


_NBLK = 8
_BROWS = _KP // _NBLK


def _nms_body(tb_col_ref, tc_col_ref, tbT_ref, tcf_ref, topv_ref,
              keep_ref, fs_ref, B_ref):
    pid = pl.program_id(0)

    @pl.when(pid < _NBLK)
    def _build():
        r0 = pid * _BROWS
        tc_c = tc_col_ref[pl.ds(r0, _BROWS), :] * 10000.0
        Lc = tb_col_ref[pl.ds(r0, _BROWS), 0:1] + tc_c
        Tc = tb_col_ref[pl.ds(r0, _BROWS), 1:2] + tc_c
        Rc = tb_col_ref[pl.ds(r0, _BROWS), 2:3] + tc_c
        Bc = tb_col_ref[pl.ds(r0, _BROWS), 3:4] + tc_c
        area_c = jnp.maximum(Rc - Lc, 0.0) * jnp.maximum(Bc - Tc, 0.0)
        tc_r = tcf_ref[...] * 10000.0
        Lr = tbT_ref[0:1, :] + tc_r
        Tr = tbT_ref[1:2, :] + tc_r
        Rr = tbT_ref[2:3, :] + tc_r
        Br = tbT_ref[3:4, :] + tc_r
        area_r = jnp.maximum(Rr - Lr, 0.0) * jnp.maximum(Br - Tr, 0.0)
        wx = jnp.clip(jnp.minimum(Rc, Rr) - jnp.maximum(Lc, Lr), 0.0, None)
        wy = jnp.clip(jnp.minimum(Bc, Br) - jnp.maximum(Tc, Tr), 0.0, None)
        inter = wx * wy
        iou = inter / (area_c + area_r - inter + 1e-9)
        ii = lax.broadcasted_iota(jnp.int32, (_BROWS, _KP), 0) + r0
        jj = lax.broadcasted_iota(jnp.int32, (_BROWS, _KP), 1)
        B_ref[pl.ds(r0, _BROWS), :] = (
            (iou > _IOU_THR) & (jj > ii)).astype(jnp.float32)

    @pl.when(pid == _NBLK)
    def _loop():
        topv = topv_ref[...]
        kv0 = (topv > 0.0).astype(jnp.float32)
        jl = lax.broadcasted_iota(jnp.int32, (1, _KP), 1)

        def body(i, kv):
            ki = jnp.sum(jnp.where(jl == i, kv, 0.0))
            row = B_ref[pl.ds(i, 1), :]
            return kv * (1.0 - row * ki)

        kv = lax.fori_loop(0, _K, body, kv0)
        keep_ref[...] = kv
        fs_ref[...] = kv * topv


def _nms(tb_col, tc_col, tbT, tcf, topv):
    full = lambda s: pl.BlockSpec(s, lambda i: (0,) * len(s))
    return pl.pallas_call(
        _nms_body,
        grid=(_NBLK + 1,),
        in_specs=[
            full((_KP, 4)),
            full((_KP, 1)),
            full((4, _KP)),
            full((1, _KP)),
            full((1, _KP)),
        ],
        out_specs=[full((1, _KP)), full((1, _KP))],
        out_shape=[
            jax.ShapeDtypeStruct((1, _KP), jnp.float32),
            jax.ShapeDtypeStruct((1, _KP), jnp.float32),
        ],
        scratch_shapes=[pltpu.VMEM((_KP, _KP), jnp.float32)],
    )(tb_col, tc_col, tbT, tcf, topv)


def kernel(prediction, frame_h, frame_w):
    pred = prediction[0]
    predT = pred.T
    scores, cid, ltrb = _phase_a(predT)
    topv, topi = lax.top_k(scores[0], _K)
    tbT = ltrb[:, topi]
    tcv = cid[0, topi]
    tbT_p = jnp.pad(tbT, ((0, 0), (0, _KP - _K)))
    tcf_row = jnp.pad(tcv.astype(jnp.float32)[None, :], ((0, 0), (0, _KP - _K)))
    topv_row = jnp.pad(topv[None, :], ((0, 0), (0, _KP - _K)),
                       constant_values=-1.0)
    keep, fs = _nms(tbT_p.T, tcf_row.T, tbT_p, tcf_row, topv_row)
    keepv = keep[0, :_K]
    det = jnp.concatenate(
        [tbT.T, fs[0, :_K, None], keepv[:, None]], axis=1)
    return det, tcv


# speedup vs baseline: 15.4708x; 2.2583x over previous
import functools

import jax
import jax.numpy as jnp
from jax import lax
from jax.experimental import pallas as pl
from jax.experimental.pallas import tpu as pltpu

_MIN_CONF = 0.25
_IOU_THR = 0.45
_N = 20000
_K = 1000
_KP = 1024
_GAIN = min(640 / 1080, 640 / 1920)
_PAD_X = (640 - 1920 * _GAIN) / 2.0
_PAD_Y = (640 - 1080 * _GAIN) / 2.0


def _phase_a_body(predT_ref, scores_ref, cid_ref, ltrb_ref):
    cls = predT_ref[5:85, :]
    m = jnp.max(cls, axis=0, keepdims=True)
    i0 = lax.broadcasted_iota(jnp.int32, (80, _N), 0)
    cid = jnp.min(jnp.where(cls == m, i0, jnp.int32(2**30)), axis=0,
                  keepdims=True)
    obj = predT_ref[4:5, :]
    conf = obj * m
    cvalid = (cid <= 3) | (cid == 5) | (cid == 7)
    keep = (conf >= _MIN_CONF) & cvalid
    scores_ref[...] = jnp.where(keep, conf, -1.0)
    cid_ref[...] = cid
    x = predT_ref[0:1, :]
    y = predT_ref[1:2, :]
    w = predT_ref[2:3, :]
    h = predT_ref[3:4, :]
    ltrb_ref[0:1, :] = jnp.round((x - w / 2.0 - _PAD_X) / _GAIN)
    ltrb_ref[1:2, :] = jnp.round((y - h / 2.0 - _PAD_Y) / _GAIN)
    ltrb_ref[2:3, :] = jnp.round((x + w / 2.0 - _PAD_X) / _GAIN)
    ltrb_ref[3:4, :] = jnp.round((y + h / 2.0 - _PAD_Y) / _GAIN)


def _phase_a(predT):
    return pl.pallas_call(
        _phase_a_body,
        out_shape=[
            jax.ShapeDtypeStruct((1, _N), jnp.float32),
            jax.ShapeDtypeStruct((1, _N), jnp.int32),
            jax.ShapeDtypeStruct((4, _N), jnp.float32),
        ],
    )(predT)


_NBLK = 8
_BROWS = _KP // _NBLK


def _nms_body(tb_col_ref, tc_col_ref, tbT_ref, tcf_ref, topv_ref,
              keep_ref, fs_ref, B_ref):
    pid = pl.program_id(0)

    @pl.when(pid < _NBLK)
    def _build():
        r0 = pid * _BROWS
        tc_c = tc_col_ref[pl.ds(r0, _BROWS), :] * 10000.0
        Lc = tb_col_ref[pl.ds(r0, _BROWS), 0:1] + tc_c
        Tc = tb_col_ref[pl.ds(r0, _BROWS), 1:2] + tc_c
        Rc = tb_col_ref[pl.ds(r0, _BROWS), 2:3] + tc_c
        Bc = tb_col_ref[pl.ds(r0, _BROWS), 3:4] + tc_c
        area_c = jnp.maximum(Rc - Lc, 0.0) * jnp.maximum(Bc - Tc, 0.0)
        tc_r = tcf_ref[...] * 10000.0
        Lr = tbT_ref[0:1, :] + tc_r
        Tr = tbT_ref[1:2, :] + tc_r
        Rr = tbT_ref[2:3, :] + tc_r
        Br = tbT_ref[3:4, :] + tc_r
        area_r = jnp.maximum(Rr - Lr, 0.0) * jnp.maximum(Br - Tr, 0.0)
        wx = jnp.clip(jnp.minimum(Rc, Rr) - jnp.maximum(Lc, Lr), 0.0, None)
        wy = jnp.clip(jnp.minimum(Bc, Br) - jnp.maximum(Tc, Tr), 0.0, None)
        inter = wx * wy
        iou = inter / (area_c + area_r - inter + 1e-9)
        ii = lax.broadcasted_iota(jnp.int32, (_BROWS, _KP), 0) + r0
        jj = lax.broadcasted_iota(jnp.int32, (_BROWS, _KP), 1)
        B_ref[pl.ds(r0, _BROWS), :] = (
            (iou > _IOU_THR) & (jj > ii)).astype(jnp.float32)

    @pl.when(pid == _NBLK)
    def _loop():
        topv = topv_ref[...]
        valid = (topv > 0.0).astype(jnp.float32)

        def cond(c):
            _, ch, t = c
            return ch & (t < _K)

        def body(c):
            kv, _, t = c
            sup = jax.lax.dot_general(
                kv, B_ref[...], (((1,), (0,)), ((), ())),
                preferred_element_type=jnp.float32)
            nk = valid * (sup == 0.0).astype(jnp.float32)
            return nk, jnp.any(nk != kv), t + 1

        kv, _, _ = lax.while_loop(
            cond, body, (valid, jnp.bool_(True), jnp.int32(0)))
        keep_ref[...] = kv
        fs_ref[...] = kv * topv


def _nms(tb_col, tc_col, tbT, tcf, topv):
    full = lambda s: pl.BlockSpec(s, lambda i: (0,) * len(s))
    return pl.pallas_call(
        _nms_body,
        grid=(_NBLK + 1,),
        in_specs=[
            full((_KP, 4)),
            full((_KP, 1)),
            full((4, _KP)),
            full((1, _KP)),
            full((1, _KP)),
        ],
        out_specs=[full((1, _KP)), full((1, _KP))],
        out_shape=[
            jax.ShapeDtypeStruct((1, _KP), jnp.float32),
            jax.ShapeDtypeStruct((1, _KP), jnp.float32),
        ],
        scratch_shapes=[pltpu.VMEM((_KP, _KP), jnp.float32)],
    )(tb_col, tc_col, tbT, tcf, topv)


def kernel(prediction, frame_h, frame_w):
    pred = prediction[0]
    predT = pred.T
    scores, cid, ltrb = _phase_a(predT)
    topv, topi = lax.top_k(scores[0], _K)
    tbT = ltrb[:, topi]
    tcv = cid[0, topi]
    tbT_p = jnp.pad(tbT, ((0, 0), (0, _KP - _K)))
    tcf_row = jnp.pad(tcv.astype(jnp.float32)[None, :], ((0, 0), (0, _KP - _K)))
    topv_row = jnp.pad(topv[None, :], ((0, 0), (0, _KP - _K)),
                       constant_values=-1.0)
    keep, fs = _nms(tbT_p.T, tcf_row.T, tbT_p, tcf_row, topv_row)
    keepv = keep[0, :_K]
    det = jnp.concatenate(
        [tbT.T, fs[0, :_K, None], keepv[:, None]], axis=1)
    return det, tcv
